# Initial kernel scaffold; baseline (speedup 1.0000x reference)
import jax
import jax.numpy as jnp
from jax.experimental import pallas as pl


def kernel(embeddings, mask_token, shuffled_indices, skip):
    raise NotImplementedError("write your pallas kernel here")


# speedup vs baseline: 5.1349x; 5.1349x over previous
import jax
import jax.numpy as jnp
from jax.experimental import pallas as pl
from jax.experimental.pallas import tpu as pltpu


def _mask_kernel(bm_ref, kn_ref, s_ref, emb_ref, mt_ref, out_ref, keep_ref):
    b = pl.program_id(0)
    tb = pl.program_id(1)
    n = keep_ref.shape[0]

    @pl.when(jnp.logical_and(b == 0, tb == 0))
    def _compute_keep():
        s_row = s_ref[...]
        i_row = jax.lax.broadcasted_iota(jnp.int32, (1, n), 1)
        valid = (i_row < kn_ref[0]).astype(jnp.int32)
        t_col = jax.lax.broadcasted_iota(jnp.int32, (n, 1), 0)
        hit = jnp.where(s_row == t_col, valid, 0)
        keep_ref[...] = jnp.max(hit, axis=1, keepdims=True)

    t_blk = out_ref.shape[1]
    keep_blk = keep_ref[pl.ds(tb * t_blk, t_blk), :]
    out_ref[0] = jnp.where(keep_blk != 0, emb_ref[0], mt_ref[0, 0][None, :])


def kernel(embeddings, mask_token, shuffled_indices, skip):
    B, N, D = embeddings.shape
    n = shuffled_indices.shape[0]
    T = 256 if n % 256 == 0 else n
    TB = n // T

    keep_n = jnp.asarray(n - skip, dtype=jnp.int32).reshape(1)
    s2d = shuffled_indices.astype(jnp.int32).reshape(1, n)

    idx = jnp.arange(n, dtype=jnp.int32)
    in_blk = shuffled_indices.astype(jnp.int32) // T
    is_kept = (idx < keep_n[0]).astype(jnp.int32)
    counts = jnp.sum(
        jnp.where(in_blk[:, None] == jnp.arange(TB, dtype=jnp.int32)[None, :],
                  is_kept[:, None], 0),
        axis=0)
    bm = jax.lax.cummax(jnp.where(counts > 0, jnp.arange(TB, dtype=jnp.int32), 0))

    grid_spec = pltpu.PrefetchScalarGridSpec(
        num_scalar_prefetch=2,
        grid=(B, TB),
        in_specs=[
            pl.BlockSpec((1, n), lambda b, tb, bm, kn: (0, 0)),
            pl.BlockSpec((1, T, D), lambda b, tb, bm, kn: (b, bm[tb], 0)),
            pl.BlockSpec((1, 1, D), lambda b, tb, bm, kn: (0, 0, 0)),
        ],
        out_specs=pl.BlockSpec((1, T, D), lambda b, tb, bm, kn: (b, tb, 0)),
        scratch_shapes=[pltpu.VMEM((n, 1), jnp.int32)],
    )

    return pl.pallas_call(
        _mask_kernel,
        grid_spec=grid_spec,
        out_shape=jax.ShapeDtypeStruct((B, N, D), embeddings.dtype),
    )(bm, keep_n, s2d, embeddings, mask_token)


# speedup vs baseline: 9.1890x; 1.7895x over previous
import jax
import jax.numpy as jnp
from jax.experimental import pallas as pl
from jax.experimental.pallas import tpu as pltpu


def _mask_kernel(bm_ref, kn_ref, s_ref, emb_ref, mt_ref, out_ref, keep_ref):
    b = pl.program_id(0)
    tb = pl.program_id(1)
    n = keep_ref.shape[0]

    @pl.when(jnp.logical_and(b == 0, tb == 0))
    def _compute_keep():
        s_row = s_ref[...]
        i_row = jax.lax.broadcasted_iota(jnp.int32, (1, n), 1)
        valid = (i_row < kn_ref[0]).astype(jnp.int32)
        t_col = jax.lax.broadcasted_iota(jnp.int32, (n, 1), 0)
        hit = jnp.where(s_row == t_col, valid, 0)
        keep_ref[...] = jnp.max(hit, axis=1, keepdims=True)

    t_blk = out_ref.shape[1]
    keep_blk = keep_ref[pl.ds(tb * t_blk, t_blk), :]
    out_ref[...] = jnp.where(keep_blk[None, :, :] != 0, emb_ref[...],
                             mt_ref[0, 0][None, None, :])


def kernel(embeddings, mask_token, shuffled_indices, skip):
    B, N, D = embeddings.shape
    n = shuffled_indices.shape[0]
    T = 256 if n % 256 == 0 else n
    TB = n // T
    BB = 4 if B % 4 == 0 else 1

    keep_n = jnp.asarray(n - skip, dtype=jnp.int32).reshape(1)
    s2d = shuffled_indices.astype(jnp.int32).reshape(1, n)

    idx = jnp.arange(n, dtype=jnp.int32)
    in_blk = shuffled_indices.astype(jnp.int32) // T
    is_kept = (idx < keep_n[0]).astype(jnp.int32)
    counts = jnp.sum(
        jnp.where(in_blk[:, None] == jnp.arange(TB, dtype=jnp.int32)[None, :],
                  is_kept[:, None], 0),
        axis=0)
    bm = jax.lax.cummax(jnp.where(counts > 0, jnp.arange(TB, dtype=jnp.int32), 0))

    grid_spec = pltpu.PrefetchScalarGridSpec(
        num_scalar_prefetch=2,
        grid=(B // BB, TB),
        in_specs=[
            pl.BlockSpec((1, n), lambda b, tb, bm, kn: (0, 0)),
            pl.BlockSpec((BB, T, D), lambda b, tb, bm, kn: (b, bm[tb], 0)),
            pl.BlockSpec((1, 1, D), lambda b, tb, bm, kn: (0, 0, 0)),
        ],
        out_specs=pl.BlockSpec((BB, T, D), lambda b, tb, bm, kn: (b, tb, 0)),
        scratch_shapes=[pltpu.VMEM((n, 1), jnp.int32)],
    )

    return pl.pallas_call(
        _mask_kernel,
        grid_spec=grid_spec,
        out_shape=jax.ShapeDtypeStruct((B, N, D), embeddings.dtype),
    )(bm, keep_n, s2d, embeddings, mask_token)


# speedup vs baseline: 10.4102x; 1.1329x over previous
import jax
import jax.numpy as jnp
from jax.experimental import pallas as pl
from jax.experimental.pallas import tpu as pltpu


def _mask_kernel(bm_ref, kn_ref, s_ref, emb_ref, mt_ref, out_ref, keep_ref):
    b = pl.program_id(0)
    tb = pl.program_id(1)
    n = keep_ref.shape[0]

    @pl.when(jnp.logical_and(b == 0, tb == 0))
    def _compute_keep():
        s_row = s_ref[...]
        i_row = jax.lax.broadcasted_iota(jnp.int32, (1, n), 1)
        valid = (i_row < kn_ref[0]).astype(jnp.int32)
        t_col = jax.lax.broadcasted_iota(jnp.int32, (n, 1), 0)
        hit = jnp.where(s_row == t_col, valid, 0)
        keep_ref[...] = jnp.max(hit, axis=1, keepdims=True)

    t_blk = out_ref.shape[1]
    keep_blk = keep_ref[pl.ds(tb * t_blk, t_blk), :]
    out_ref[...] = jnp.where(keep_blk[None, :, :] != 0, emb_ref[...],
                             mt_ref[0, 0][None, None, :])


def kernel(embeddings, mask_token, shuffled_indices, skip):
    B, N, D = embeddings.shape
    n = shuffled_indices.shape[0]
    T = 256 if n % 256 == 0 else n
    TB = n // T
    BB = 8 if B % 8 == 0 else (4 if B % 4 == 0 else 1)

    keep_n = jnp.asarray(n - skip, dtype=jnp.int32).reshape(1)
    s2d = shuffled_indices.astype(jnp.int32).reshape(1, n)

    idx = jnp.arange(n, dtype=jnp.int32)
    in_blk = shuffled_indices.astype(jnp.int32) // T
    is_kept = (idx < keep_n[0]).astype(jnp.int32)
    counts = jnp.sum(
        jnp.where(in_blk[:, None] == jnp.arange(TB, dtype=jnp.int32)[None, :],
                  is_kept[:, None], 0),
        axis=0)
    bm = jax.lax.cummax(jnp.where(counts > 0, jnp.arange(TB, dtype=jnp.int32), 0))

    grid_spec = pltpu.PrefetchScalarGridSpec(
        num_scalar_prefetch=2,
        grid=(B // BB, TB),
        in_specs=[
            pl.BlockSpec((1, n), lambda b, tb, bm, kn: (0, 0)),
            pl.BlockSpec((BB, T, D), lambda b, tb, bm, kn: (b, bm[tb], 0)),
            pl.BlockSpec((1, 1, D), lambda b, tb, bm, kn: (0, 0, 0)),
        ],
        out_specs=pl.BlockSpec((BB, T, D), lambda b, tb, bm, kn: (b, tb, 0)),
        scratch_shapes=[pltpu.VMEM((n, 1), jnp.int32)],
    )

    return pl.pallas_call(
        _mask_kernel,
        grid_spec=grid_spec,
        out_shape=jax.ShapeDtypeStruct((B, N, D), embeddings.dtype),
    )(bm, keep_n, s2d, embeddings, mask_token)


# speedup vs baseline: 11.1219x; 1.0684x over previous
import jax
import jax.numpy as jnp
from jax.experimental import pallas as pl
from jax.experimental.pallas import tpu as pltpu


def _mask_kernel(bm_ref, kn_ref, s_ref, emb_ref, mt_ref, out_ref, keep_ref):
    b = pl.program_id(0)
    tb = pl.program_id(1)
    n = keep_ref.shape[0]

    @pl.when(jnp.logical_and(b == 0, tb == 0))
    def _compute_keep():
        s_row = s_ref[...]
        i_row = jax.lax.broadcasted_iota(jnp.int32, (1, n), 1)
        valid = (i_row < kn_ref[0]).astype(jnp.int32)
        t_col = jax.lax.broadcasted_iota(jnp.int32, (n, 1), 0)
        hit = jnp.where(s_row == t_col, valid, 0)
        keep_ref[...] = jnp.max(hit, axis=1, keepdims=True)

    t_blk = out_ref.shape[1]
    keep_blk = keep_ref[pl.ds(tb * t_blk, t_blk), :]
    out_ref[...] = jnp.where(keep_blk[None, :, :] != 0, emb_ref[...],
                             mt_ref[0, 0][None, None, :])


def kernel(embeddings, mask_token, shuffled_indices, skip):
    B, N, D = embeddings.shape
    n = shuffled_indices.shape[0]
    T = 256 if n % 256 == 0 else n
    TB = n // T
    BB = 16 if B % 16 == 0 else (4 if B % 4 == 0 else 1)

    keep_n = jnp.asarray(n - skip, dtype=jnp.int32).reshape(1)
    s2d = shuffled_indices.astype(jnp.int32).reshape(1, n)

    idx = jnp.arange(n, dtype=jnp.int32)
    in_blk = shuffled_indices.astype(jnp.int32) // T
    is_kept = (idx < keep_n[0]).astype(jnp.int32)
    counts = jnp.sum(
        jnp.where(in_blk[:, None] == jnp.arange(TB, dtype=jnp.int32)[None, :],
                  is_kept[:, None], 0),
        axis=0)
    bm = jax.lax.cummax(jnp.where(counts > 0, jnp.arange(TB, dtype=jnp.int32), 0))

    grid_spec = pltpu.PrefetchScalarGridSpec(
        num_scalar_prefetch=2,
        grid=(B // BB, TB),
        in_specs=[
            pl.BlockSpec((1, n), lambda b, tb, bm, kn: (0, 0)),
            pl.BlockSpec((BB, T, D), lambda b, tb, bm, kn: (b, bm[tb], 0)),
            pl.BlockSpec((1, 1, D), lambda b, tb, bm, kn: (0, 0, 0)),
        ],
        out_specs=pl.BlockSpec((BB, T, D), lambda b, tb, bm, kn: (b, tb, 0)),
        scratch_shapes=[pltpu.VMEM((n, 1), jnp.int32)],
    )

    return pl.pallas_call(
        _mask_kernel,
        grid_spec=grid_spec,
        out_shape=jax.ShapeDtypeStruct((B, N, D), embeddings.dtype),
    )(bm, keep_n, s2d, embeddings, mask_token)
